# baseline (device time: 112729 ns/iter reference)
import functools

import jax
import jax.numpy as jnp
from jax import lax
from jax.experimental import pallas as pl
from jax.experimental.pallas import tpu as pltpu

N_DEV = 32
N_STAGES = 5


def _mlp_layer_allreduce(x, win, wout, *, collective_id):
    b, d = x.shape

    def body(x_ref, win_ref, wout_ref, out_ref,
             send_buf, recv_buf, send_sem, recv_sems):
        my = lax.axis_index("i")

        barrier_sem = pltpu.get_barrier_semaphore()
        for k in range(N_STAGES):
            partner = my ^ (1 << k)
            pl.semaphore_signal(
                barrier_sem, inc=1,
                device_id=(partner,), device_id_type=pl.DeviceIdType.MESH,
            )

        h = jnp.maximum(
            jnp.dot(x_ref[...], win_ref[...],
                    preferred_element_type=jnp.float32),
            0.0,
        )
        acc = jnp.dot(h, wout_ref[...], preferred_element_type=jnp.float32)

        pl.semaphore_wait(barrier_sem, N_STAGES)

        for k in range(N_STAGES):
            partner = my ^ (1 << k)
            send_buf[...] = acc
            rdma = pltpu.make_async_remote_copy(
                src_ref=send_buf,
                dst_ref=recv_buf.at[k],
                send_sem=send_sem,
                recv_sem=recv_sems.at[k],
                device_id=(partner,),
                device_id_type=pl.DeviceIdType.MESH,
            )
            rdma.start()
            rdma.wait()
            acc = acc + recv_buf[k]

        out_ref[...] = acc

    return pl.pallas_call(
        body,
        out_shape=jax.ShapeDtypeStruct((b, d), jnp.float32),
        in_specs=[
            pl.BlockSpec(memory_space=pltpu.VMEM),
            pl.BlockSpec(memory_space=pltpu.VMEM),
            pl.BlockSpec(memory_space=pltpu.VMEM),
        ],
        out_specs=pl.BlockSpec(memory_space=pltpu.VMEM),
        scratch_shapes=[
            pltpu.VMEM((b, d), jnp.float32),
            pltpu.VMEM((N_STAGES, b, d), jnp.float32),
            pltpu.SemaphoreType.DMA,
            pltpu.SemaphoreType.DMA((N_STAGES,)),
        ],
        compiler_params=pltpu.CompilerParams(collective_id=collective_id),
    )(x, win, wout)


def kernel(x, Win0, Wout0, Win1, Wout1, Win2, Wout2):
    x = _mlp_layer_allreduce(x, Win0, Wout0, collective_id=0)
    x = _mlp_layer_allreduce(x, Win1, Wout1, collective_id=1)
    x = _mlp_layer_allreduce(x, Win2, Wout2, collective_id=2)
    return x


# device time: 82791 ns/iter; 1.3616x vs baseline; 1.3616x over previous
import functools

import jax
import jax.numpy as jnp
from jax import lax
from jax.experimental import pallas as pl
from jax.experimental.pallas import tpu as pltpu

N_DEV = 32
N_STAGES = 5


def _mlp_layer_allreduce(x, win, wout, *, collective_id):
    b, d = x.shape

    def body(x_ref, win_ref, wout_ref, out_ref,
             send_buf, recv_buf, send_sem, recv_sems):
        my = lax.axis_index("i")

        barrier_sem = pltpu.get_barrier_semaphore()
        for k in range(N_STAGES):
            partner = my ^ (1 << k)
            pl.semaphore_signal(
                barrier_sem, inc=1,
                device_id=(partner,), device_id_type=pl.DeviceIdType.MESH,
            )

        h = jnp.maximum(
            jnp.dot(x_ref[...], win_ref[...],
                    preferred_element_type=jnp.float32),
            0.0,
        )
        acc = jnp.dot(h, wout_ref[...], preferred_element_type=jnp.float32)

        pl.semaphore_wait(barrier_sem, N_STAGES)

        for k in range(N_STAGES):
            partner = my ^ (1 << k)
            send_buf[...] = acc.astype(jnp.bfloat16)
            rdma = pltpu.make_async_remote_copy(
                src_ref=send_buf,
                dst_ref=recv_buf.at[k],
                send_sem=send_sem,
                recv_sem=recv_sems.at[k],
                device_id=(partner,),
                device_id_type=pl.DeviceIdType.MESH,
            )
            rdma.start()
            rdma.wait()
            acc = acc + recv_buf[k].astype(jnp.float32)

        out_ref[...] = acc

    return pl.pallas_call(
        body,
        out_shape=jax.ShapeDtypeStruct((b, d), jnp.float32),
        in_specs=[
            pl.BlockSpec(memory_space=pltpu.VMEM),
            pl.BlockSpec(memory_space=pltpu.VMEM),
            pl.BlockSpec(memory_space=pltpu.VMEM),
        ],
        out_specs=pl.BlockSpec(memory_space=pltpu.VMEM),
        scratch_shapes=[
            pltpu.VMEM((b, d), jnp.bfloat16),
            pltpu.VMEM((N_STAGES, b, d), jnp.bfloat16),
            pltpu.SemaphoreType.DMA,
            pltpu.SemaphoreType.DMA((N_STAGES,)),
        ],
        compiler_params=pltpu.CompilerParams(collective_id=collective_id),
    )(x, win, wout)


def kernel(x, Win0, Wout0, Win1, Wout1, Win2, Wout2):
    x = _mlp_layer_allreduce(x, Win0, Wout0, collective_id=0)
    x = _mlp_layer_allreduce(x, Win1, Wout1, collective_id=1)
    x = _mlp_layer_allreduce(x, Win2, Wout2, collective_id=2)
    return x


# device time: 73754 ns/iter; 1.5284x vs baseline; 1.1225x over previous
import jax
import jax.numpy as jnp
from jax import lax
from jax.experimental import pallas as pl
from jax.experimental.pallas import tpu as pltpu

N_DEV = 32
N_STAGES = 5
N_LAYERS = 3


def kernel(x, Win0, Wout0, Win1, Wout1, Win2, Wout2):
    b, d = x.shape
    h_per = Win0.shape[1]

    def body(x_ref, win0_ref, wout0_ref, win1_ref, wout1_ref,
             win2_ref, wout2_ref, out_ref,
             win_buf, wout_buf, send_buf, recv_buf,
             copy_sems, send_sem, recv_sems):
        my = lax.axis_index("i")

        z = my // 8
        q = my % 8
        y = q // 2
        x_c = (q % 2) ^ (y & 1)

        def pos(xc, yc, zc):
            return 8 * zc + 2 * yc + (xc ^ (yc & 1))

        partners = [
            pos(1 - x_c, y, z),
            pos(x_c, y ^ 1, z),
            pos(x_c, y, z ^ 1),
            pos(x_c, y ^ 2, z),
            pos(x_c, y, z ^ 2),
        ]

        wins = [win0_ref, win1_ref, win2_ref]
        wouts = [wout0_ref, wout1_ref, wout2_ref]

        def start_weight_copies(layer, slot):
            win_cp = pltpu.make_async_copy(
                wins[layer], win_buf.at[slot], copy_sems.at[layer, 0])
            wout_cp = pltpu.make_async_copy(
                wouts[layer], wout_buf.at[slot], copy_sems.at[layer, 1])
            win_cp.start()
            wout_cp.start()
            return win_cp, wout_cp

        cps = start_weight_copies(0, 0)

        barrier_sem = pltpu.get_barrier_semaphore()
        for partner in partners:
            pl.semaphore_signal(
                barrier_sem, inc=1,
                device_id=(partner,), device_id_type=pl.DeviceIdType.MESH,
            )

        acc = x_ref[...]
        for layer in range(N_LAYERS):
            slot = layer % 2
            win_cp, wout_cp = cps
            win_cp.wait()
            h = jnp.maximum(
                jnp.dot(acc, win_buf[slot],
                        preferred_element_type=jnp.float32),
                0.0,
            )
            wout_cp.wait()
            acc = jnp.dot(h, wout_buf[slot],
                          preferred_element_type=jnp.float32)

            if layer + 1 < N_LAYERS:
                cps = start_weight_copies(layer + 1, 1 - slot)

            if layer == 0:
                pl.semaphore_wait(barrier_sem, N_STAGES)

            for k, partner in enumerate(partners):
                send_buf[...] = acc.astype(jnp.bfloat16)
                rdma = pltpu.make_async_remote_copy(
                    src_ref=send_buf,
                    dst_ref=recv_buf.at[layer, k],
                    send_sem=send_sem,
                    recv_sem=recv_sems.at[layer, k],
                    device_id=(partner,),
                    device_id_type=pl.DeviceIdType.MESH,
                )
                rdma.start()
                rdma.wait()
                acc = acc + recv_buf[layer, k].astype(jnp.float32)

        out_ref[...] = acc

    return pl.pallas_call(
        body,
        out_shape=jax.ShapeDtypeStruct((b, d), jnp.float32),
        in_specs=[pl.BlockSpec(memory_space=pltpu.VMEM)]
        + [pl.BlockSpec(memory_space=pltpu.MemorySpace.HBM)] * 6,
        out_specs=pl.BlockSpec(memory_space=pltpu.VMEM),
        scratch_shapes=[
            pltpu.VMEM((2, d, h_per), jnp.float32),
            pltpu.VMEM((2, h_per, d), jnp.float32),
            pltpu.VMEM((b, d), jnp.bfloat16),
            pltpu.VMEM((N_LAYERS, N_STAGES, b, d), jnp.bfloat16),
            pltpu.SemaphoreType.DMA((N_LAYERS, 2)),
            pltpu.SemaphoreType.DMA,
            pltpu.SemaphoreType.DMA((N_LAYERS, N_STAGES)),
        ],
        compiler_params=pltpu.CompilerParams(
            collective_id=0, vmem_limit_bytes=64 * 1024 * 1024),
    )(x, Win0, Wout0, Win1, Wout1, Win2, Wout2)
